# speedup vs baseline: 1.6585x; 1.6585x over previous
import functools
import math

import jax
import jax.numpy as jnp
from jax import lax
from jax.experimental import pallas as pl
from jax.experimental.pallas import tpu as pltpu
from jax.experimental.pallas import tpu_sc as plsc

NINP = 128
SEQ = 200
BATCH = 1024
ROWS = SEQ * BATCH
NWORKERS = 32
RPW = ROWS // NWORKERS
STAGE = 128
NSTAGES = RPW // STAGE
LANES = 16
_SCALE = math.sqrt(NINP)


def _pe_table():
    position = jnp.arange(0, SEQ, dtype=jnp.float32)[:, None]
    div_term = jnp.exp(
        jnp.arange(0, NINP, 2, dtype=jnp.float32) * (-math.log(10000.0) / NINP)
    )
    pe = jnp.zeros((SEQ, NINP), dtype=jnp.float32)
    pe = pe.at[:, 0::2].set(jnp.sin(position * div_term))
    pe = pe.at[:, 1::2].set(jnp.cos(position * div_term))
    return pe


@functools.partial(
    pl.kernel,
    mesh=plsc.VectorSubcoreMesh(core_axis_name="c", subcore_axis_name="s"),
    out_type=jax.ShapeDtypeStruct((ROWS, NINP), jnp.float32),
    scratch_types=[
        pltpu.VMEM((RPW,), jnp.int32),
        pltpu.VMEM((SEQ, NINP), jnp.float32),
        pltpu.VMEM((STAGE, NINP), jnp.float32),
        pltpu.VMEM((STAGE, NINP), jnp.float32),
        pltpu.VMEM((STAGE, NINP), jnp.float32),
        pltpu.VMEM((STAGE, NINP), jnp.float32),
        pltpu.SemaphoreType.DMA,
        pltpu.SemaphoreType.DMA,
        pltpu.SemaphoreType.DMA,
        pltpu.SemaphoreType.DMA,
    ],
)
def _encode_sc(idx_hbm, table_hbm, pe_hbm, out_hbm,
               idx_v, pe_v, g0, g1, o0, o1, gsem0, gsem1, wsem0, wsem1):
    wid = lax.axis_index("s") * 2 + lax.axis_index("c")
    base = wid * RPW

    pltpu.sync_copy(idx_hbm.at[pl.ds(base, RPW)], idx_v)
    pltpu.sync_copy(pe_hbm, pe_v)

    def start_gather(t, gbuf, gsem):
        pltpu.async_copy(
            table_hbm.at[idx_v.at[pl.ds(t * STAGE, STAGE)]], gbuf, gsem)

    def wait_gather(gbuf, gsem):
        pltpu.make_async_copy(table_hbm.at[pl.ds(0, STAGE)], gbuf, gsem).wait()

    def wait_write(obuf, wsem):
        pltpu.make_async_copy(obuf, out_hbm.at[pl.ds(0, STAGE)], wsem).wait()

    def compute(t, gbuf, obuf):
        s = (base + t * STAGE) // BATCH
        pe_rows = [pe_v[s, pl.ds(LANES * j, LANES)] for j in range(NINP // LANES)]

        def row(r, carry):
            for j in range(NINP // LANES):
                sl = pl.ds(LANES * j, LANES)
                obuf[r, sl] = gbuf[r, sl] * _SCALE + pe_rows[j]
            return carry

        lax.fori_loop(0, STAGE, row, 0)

    start_gather(0, g0, gsem0)
    start_gather(1, g1, gsem1)

    def outer(i, carry):
        for p, (gbuf, obuf, gsem, wsem) in enumerate(
                ((g0, o0, gsem0, wsem0), (g1, o1, gsem1, wsem1))):
            t = 2 * i + p
            wait_gather(gbuf, gsem)

            @pl.when(i > 0)
            def _():
                wait_write(obuf, wsem)

            compute(t, gbuf, obuf)

            @pl.when(t + 2 < NSTAGES)
            def _():
                start_gather(t + 2, gbuf, gsem)

            pltpu.async_copy(obuf, out_hbm.at[pl.ds(base + t * STAGE, STAGE)],
                             wsem)
        return carry

    lax.fori_loop(0, NSTAGES // 2, outer, 0)
    wait_write(o0, wsem0)
    wait_write(o1, wsem1)


def kernel(src, emb_weight):
    idx = src.T.reshape(-1).astype(jnp.int32)
    out = _encode_sc(idx, emb_weight, _pe_table())
    return out.reshape(SEQ, BATCH, NINP)


# speedup vs baseline: 1.7265x; 1.0410x over previous
import functools
import math

import jax
import jax.numpy as jnp
from jax import lax
from jax.experimental import pallas as pl
from jax.experimental.pallas import tpu as pltpu
from jax.experimental.pallas import tpu_sc as plsc

NINP = 128
SEQ = 200
BATCH = 1024
ROWS = SEQ * BATCH
NWORKERS = 32
RPW = ROWS // NWORKERS
STAGE = 128
NSTAGES = RPW // STAGE
NPHASE = 3
SROWS = 16
LANES = 16
_SCALE = math.sqrt(NINP)


def _pe_table():
    position = jnp.arange(0, SEQ, dtype=jnp.float32)[:, None]
    div_term = jnp.exp(
        jnp.arange(0, NINP, 2, dtype=jnp.float32) * (-math.log(10000.0) / NINP)
    )
    pe = jnp.zeros((SEQ, NINP), dtype=jnp.float32)
    pe = pe.at[:, 0::2].set(jnp.sin(position * div_term))
    pe = pe.at[:, 1::2].set(jnp.cos(position * div_term))
    return pe


@functools.partial(
    pl.kernel,
    mesh=plsc.VectorSubcoreMesh(core_axis_name="c", subcore_axis_name="s"),
    out_type=jax.ShapeDtypeStruct((ROWS, NINP), jnp.float32),
    scratch_types=[
        pltpu.VMEM((RPW,), jnp.int32),
        pltpu.VMEM((SROWS, NINP), jnp.float32),
        pltpu.VMEM((STAGE, NINP), jnp.float32),
        pltpu.VMEM((STAGE, NINP), jnp.float32),
        pltpu.VMEM((STAGE, NINP), jnp.float32),
        pltpu.VMEM((STAGE, NINP), jnp.float32),
        pltpu.VMEM((STAGE, NINP), jnp.float32),
        pltpu.VMEM((STAGE, NINP), jnp.float32),
        pltpu.SemaphoreType.DMA,
        pltpu.SemaphoreType.DMA,
        pltpu.SemaphoreType.DMA,
        pltpu.SemaphoreType.DMA,
        pltpu.SemaphoreType.DMA,
        pltpu.SemaphoreType.DMA,
    ],
)
def _encode_sc(idx_hbm, table_hbm, pe_hbm, out_hbm,
               idx_v, pe_v, g0, g1, g2, o0, o1, o2,
               gsem0, gsem1, gsem2, wsem0, wsem1, wsem2):
    wid = lax.axis_index("s") * 2 + lax.axis_index("c")
    base = wid * RPW
    s_lo = base // BATCH
    s_lo8 = pl.multiple_of(
        jnp.minimum(s_lo - lax.rem(s_lo, 8), SEQ - SROWS), 8)

    pltpu.sync_copy(idx_hbm.at[pl.ds(base, RPW)], idx_v)
    pltpu.sync_copy(pe_hbm.at[pl.ds(s_lo8, SROWS)], pe_v)

    def start_gather(t, gbuf, gsem):
        pltpu.async_copy(
            table_hbm.at[idx_v.at[pl.ds(t * STAGE, STAGE)]], gbuf, gsem)

    def wait_gather(gbuf, gsem):
        pltpu.make_async_copy(table_hbm.at[pl.ds(0, STAGE)], gbuf, gsem).wait()

    def wait_write(obuf, wsem):
        pltpu.make_async_copy(obuf, out_hbm.at[pl.ds(0, STAGE)], wsem).wait()

    def compute(t, gbuf, obuf):
        ds = (base + t * STAGE) // BATCH - s_lo8
        pe_rows = [pe_v[ds, pl.ds(LANES * j, LANES)]
                   for j in range(NINP // LANES)]

        def row(r, carry):
            for j in range(NINP // LANES):
                sl = pl.ds(LANES * j, LANES)
                obuf[r, sl] = gbuf[r, sl] * _SCALE + pe_rows[j]
            return carry

        lax.fori_loop(0, STAGE, row, 0)

    phases = ((g0, o0, gsem0, wsem0), (g1, o1, gsem1, wsem1),
              (g2, o2, gsem2, wsem2))

    for t in range(NPHASE):
        start_gather(t, phases[t][0], phases[t][2])

    def outer(i, carry):
        for p, (gbuf, obuf, gsem, wsem) in enumerate(phases):
            t = NPHASE * i + p
            wait_gather(gbuf, gsem)

            @pl.when(i > 0)
            def _():
                wait_write(obuf, wsem)

            compute(t, gbuf, obuf)

            @pl.when(t + NPHASE < NSTAGES)
            def _():
                start_gather(t + NPHASE, gbuf, gsem)

            pltpu.async_copy(obuf, out_hbm.at[pl.ds(base + t * STAGE, STAGE)],
                             wsem)
        return carry

    lax.fori_loop(0, NSTAGES // NPHASE, outer, 0)
    for p in range(NSTAGES % NPHASE):
        t = (NSTAGES // NPHASE) * NPHASE + p
        gbuf, obuf, gsem, wsem = phases[p]
        wait_gather(gbuf, gsem)
        wait_write(obuf, wsem)
        compute(t, gbuf, obuf)
        pltpu.async_copy(obuf, out_hbm.at[pl.ds(base + t * STAGE, STAGE)],
                         wsem)
    for p, (gbuf, obuf, gsem, wsem) in enumerate(phases):
        wait_write(obuf, wsem)


def kernel(src, emb_weight):
    idx = src.T.reshape(-1).astype(jnp.int32)
    out = _encode_sc(idx, emb_weight, _pe_table())
    return out.reshape(SEQ, BATCH, NINP)
